# baseline (speedup 1.0000x reference)
import functools

import jax
import jax.numpy as jnp
from jax import lax
from jax.experimental import pallas as pl
from jax.experimental.pallas import tpu as pltpu
from jax.experimental.pallas import tpu_sc as plsc

_L = 199
_D = 128
_NM = 32
_B = 1024
_NC = 2
_NS = 16
_NW = _NC * _NS
_BPW = _B // _NW
_NCH = _D // 16


def _w_table_body(masks_ref, ind_ref, w_ref):
    ind = ind_ref[...]
    m_iota = lax.broadcasted_iota(jnp.int32, (_NM, _D), 0)
    sel = (ind == m_iota).astype(jnp.float32)
    w_ref[...] = lax.dot_general(sel, masks_ref[...],
                                 dimension_numbers=(((0,), (0,)), ((), ())),
                                 preferred_element_type=jnp.float32)


def _sc_body(xt_hbm, xp_hbm, w_hbm, out_hbm,
             xo_0, xo_1, xp_0, xp_1, w_v,
             s_xt0, s_xt1, s_xp0, s_xp1, s_o0, s_o1):
    wid = lax.axis_index("s") * _NC + lax.axis_index("c")
    base = wid * _BPW
    pltpu.sync_copy(w_hbm, w_v)
    lanes = lax.iota(jnp.int32, 16)

    xo = (xo_0, xo_1)
    xpv = (xp_0, xp_1)
    s_xt = (s_xt0, s_xt1)
    s_xp = (s_xp0, s_xp1)
    s_o = (s_o0, s_o1)

    def in_xt(b, p):
        return pltpu.make_async_copy(xt_hbm.at[b], xo[p], s_xt[p])

    def in_xp(b, p):
        return pltpu.make_async_copy(xp_hbm.at[b], xpv[p], s_xp[p])

    def out_store(b, p):
        return pltpu.make_async_copy(xo[p], out_hbm.at[b], s_o[p])

    def make_token_fn(xt_v, xp_v):
        def one_token(t, carry):
            maxv = xt_v[t, pl.ds(0, 16)]
            cidx = jnp.zeros((16,), jnp.int32)
            for c in range(1, _NCH):
                v = xt_v[t, pl.ds(c * 16, 16)]
                better = v > maxv
                maxv = jnp.where(better, v, maxv)
                cidx = jnp.where(better, c, cidx)
            gmax = jax.lax.reduce_max(maxv, (0,))
            dcand = jnp.where(maxv == gmax, cidx * 16 + lanes, _D)
            bestd = jax.lax.reduce_min(dcand, (0,))
            es = []
            s = jnp.zeros((16,), jnp.float32)
            for c in range(_NCH):
                m = w_v[bestd, pl.ds(c * 16, 16)]
                p = xp_v[t, pl.ds(c * 16, 16)]
                e = jnp.exp(p) * m
                es.append(e)
                s = s + e
            total = jax.lax.reduce_sum(s, (0,))
            rinv = (jnp.ones((16,), jnp.float32)
                    / jnp.full((16,), total, jnp.float32))
            for c in range(_NCH):
                xt_v[t, pl.ds(c * 16, 16)] = es[c] * rinv
            return carry
        return one_token

    token_fns = (make_token_fn(xo_0, xp_0), make_token_fn(xo_1, xp_1))

    in_xt(base, 0).start()
    in_xp(base, 0).start()

    def one_pair(i, carry):
        for p in (0, 1):
            b = base + 2 * i + p
            q = 1 - p
            in_xt(b, p).wait()
            in_xp(b, p).wait()

            @pl.when(b + 1 < base + _BPW)
            def _():
                @pl.when(b > base)
                def _():
                    out_store(b - 1, q).wait()
                in_xt(b + 1, q).start()
                in_xp(b + 1, q).start()

            lax.fori_loop(0, _L, token_fns[p], 0)
            out_store(b, p).start()
        return carry

    lax.fori_loop(0, _BPW // 2, one_pair, 0)
    out_store(base + _BPW - 2, 0).wait()
    out_store(base + _BPW - 1, 1).wait()


def kernel(x_true, x_pred, masks, ind_of_ind):
    w = pl.pallas_call(
        _w_table_body,
        out_shape=jax.ShapeDtypeStruct((_D, _D), jnp.float32),
    )(masks, ind_of_ind.astype(jnp.int32).reshape(1, _D))
    mesh = plsc.VectorSubcoreMesh(core_axis_name="c", subcore_axis_name="s")
    f = functools.partial(
        pl.kernel,
        mesh=mesh,
        compiler_params=pltpu.CompilerParams(needs_layout_passes=False),
        out_type=jax.ShapeDtypeStruct((_B, _L, _D), jnp.float32),
        scratch_types=[
            pltpu.VMEM((_L, _D), jnp.float32),
            pltpu.VMEM((_L, _D), jnp.float32),
            pltpu.VMEM((_L, _D), jnp.float32),
            pltpu.VMEM((_L, _D), jnp.float32),
            pltpu.VMEM((_D, _D), jnp.float32),
            pltpu.SemaphoreType.DMA,
            pltpu.SemaphoreType.DMA,
            pltpu.SemaphoreType.DMA,
            pltpu.SemaphoreType.DMA,
            pltpu.SemaphoreType.DMA,
            pltpu.SemaphoreType.DMA,
        ],
    )(_sc_body)
    return f(x_true, x_pred, w)


# speedup vs baseline: 1.1071x; 1.1071x over previous
import functools

import jax
import jax.numpy as jnp
from jax import lax
from jax.experimental import pallas as pl
from jax.experimental.pallas import tpu as pltpu
from jax.experimental.pallas import tpu_sc as plsc

_L = 199
_D = 128
_NM = 32
_B = 1024
_NC = 2
_NS = 16
_NW = _NC * _NS
_BPW = _B // _NW
_NCH = _D // 16


def _w_table_body(masks_ref, ind_ref, w_ref):
    ind = ind_ref[...]
    m_iota = lax.broadcasted_iota(jnp.int32, (_NM, _D), 0)
    sel = (ind == m_iota).astype(jnp.float32)
    w_ref[...] = lax.dot_general(sel, masks_ref[...],
                                 dimension_numbers=(((0,), (0,)), ((), ())),
                                 preferred_element_type=jnp.float32)


def _sc_body(xt_hbm, xp_hbm, w_hbm, out_hbm, xt_v, xp_v, out_v, w_v):
    wid = lax.axis_index("s") * _NC + lax.axis_index("c")
    pltpu.sync_copy(w_hbm, w_v)
    lanes = lax.iota(jnp.int32, 16)

    def stage1(t):
        maxv = xt_v[t, pl.ds(0, 16)]
        cidx = jnp.zeros((16,), jnp.int32)
        for c in range(1, _NCH):
            v = xt_v[t, pl.ds(c * 16, 16)]
            better = v > maxv
            maxv = jnp.where(better, v, maxv)
            cidx = jnp.where(better, c, cidx)
        _, vs = plsc.sort_key_val(maxv, cidx * 16 + lanes, descending=True)
        return vs[0]

    def stage2(u, bd):
        es = []
        s = jnp.zeros((16,), jnp.float32)
        for c in range(_NCH):
            m = w_v[bd, pl.ds(c * 16, 16)]
            p = xp_v[u, pl.ds(c * 16, 16)]
            e = jnp.exp(p) * m
            es.append(e)
            s = s + e
        total = jax.lax.reduce_sum(s, (0,))
        rinv = (jnp.ones((16,), jnp.float32)
                / jnp.full((16,), total, jnp.float32))
        for c in range(_NCH):
            out_v[u, pl.ds(c * 16, 16)] = es[c] * rinv

    def one_token(t, bd_prev):
        bd = stage1(jnp.minimum(t, _L - 1))
        stage2(jnp.maximum(t - 1, 0), bd_prev)
        return bd

    def one_batch(b, carry):
        pltpu.sync_copy(xt_hbm.at[b], xt_v)
        pltpu.sync_copy(xp_hbm.at[b], xp_v)
        lax.fori_loop(0, _L + 1, one_token, 0)
        pltpu.sync_copy(out_v, out_hbm.at[b])
        return carry

    lax.fori_loop(wid * _BPW, (wid + 1) * _BPW, one_batch, 0)


def kernel(x_true, x_pred, masks, ind_of_ind):
    w = pl.pallas_call(
        _w_table_body,
        out_shape=jax.ShapeDtypeStruct((_D, _D), jnp.float32),
    )(masks, ind_of_ind.astype(jnp.int32).reshape(1, _D))
    mesh = plsc.VectorSubcoreMesh(core_axis_name="c", subcore_axis_name="s")
    f = functools.partial(
        pl.kernel,
        mesh=mesh,
        compiler_params=pltpu.CompilerParams(needs_layout_passes=False),
        out_type=jax.ShapeDtypeStruct((_B, _L, _D), jnp.float32),
        scratch_types=[
            pltpu.VMEM((_L, _D), jnp.float32),
            pltpu.VMEM((_L, _D), jnp.float32),
            pltpu.VMEM((_L, _D), jnp.float32),
            pltpu.VMEM((_D, _D), jnp.float32),
        ],
    )(_sc_body)
    return f(x_true, x_pred, w)


# speedup vs baseline: 1.3504x; 1.2198x over previous
import functools

import jax
import jax.numpy as jnp
from jax import lax
from jax.experimental import pallas as pl
from jax.experimental.pallas import tpu as pltpu
from jax.experimental.pallas import tpu_sc as plsc

_L = 199
_D = 128
_NM = 32
_B = 1024
_NC = 2
_NS = 16
_NW = _NC * _NS
_BPW = _B // _NW
_NCH = _D // 16


def _w_table_body(masks_ref, ind_ref, w_ref):
    ind = ind_ref[...]
    m_iota = lax.broadcasted_iota(jnp.int32, (_NM, _D), 0)
    sel = (ind == m_iota).astype(jnp.float32)
    w_ref[...] = lax.dot_general(sel, masks_ref[...],
                                 dimension_numbers=(((0,), (0,)), ((), ())),
                                 preferred_element_type=jnp.float32)


def _sc_body(xt_hbm, xp_hbm, w_hbm, out_hbm, xo_v, xp_v, w_v, s_in, s_out):
    wid = lax.axis_index("s") * _NC + lax.axis_index("c")
    base = wid * _BPW
    pltpu.sync_copy(w_hbm, w_v)
    lanes = lax.iota(jnp.int32, 16)

    def start_in(b, p):
        pltpu.make_async_copy(
            xt_hbm.at[b], xo_v.at[pl.ds(_L * p, _L)], s_in).start()
        pltpu.make_async_copy(
            xp_hbm.at[b], xp_v.at[pl.ds(_L * p, _L)], s_in).start()

    def wait_in(b):
        pltpu.make_async_copy(
            xt_hbm.at[b], xo_v.at[pl.ds(0, _L)], s_in).wait()
        pltpu.make_async_copy(
            xp_hbm.at[b], xp_v.at[pl.ds(0, _L)], s_in).wait()

    def start_out(b, p):
        pltpu.make_async_copy(
            xo_v.at[pl.ds(_L * p, _L)], out_hbm.at[b], s_out).start()

    def wait_out(b):
        pltpu.make_async_copy(
            xo_v.at[pl.ds(0, _L)], out_hbm.at[b], s_out).wait()

    def make_token_fn(par):
        row0 = par * _L

        def stage1(t):
            maxv = xo_v[row0 + t, pl.ds(0, 16)]
            cidx = jnp.zeros((16,), jnp.int32)
            for c in range(1, _NCH):
                v = xo_v[row0 + t, pl.ds(c * 16, 16)]
                better = v > maxv
                maxv = jnp.where(better, v, maxv)
                cidx = jnp.where(better, c, cidx)
            _, vs = plsc.sort_key_val(maxv, cidx * 16 + lanes,
                                      descending=True)
            return vs[0]

        def stage2(u, bd):
            es = []
            s = jnp.zeros((16,), jnp.float32)
            for c in range(_NCH):
                m = w_v[bd, pl.ds(c * 16, 16)]
                p = xp_v[row0 + u, pl.ds(c * 16, 16)]
                e = jnp.exp(p) * m
                es.append(e)
                s = s + e
            total = jax.lax.reduce_sum(s, (0,))
            rinv = (jnp.ones((16,), jnp.float32)
                    / jnp.full((16,), total, jnp.float32))
            for c in range(_NCH):
                xo_v[row0 + u, pl.ds(c * 16, 16)] = es[c] * rinv

        def one_token(t, bd_prev):
            bd = stage1(jnp.minimum(t, _L - 1))
            stage2(jnp.maximum(t - 1, 0), bd_prev)
            return bd
        return one_token

    start_in(base, 0)

    def one_batch(i, carry):
        b = base + i
        par = lax.rem(i, 2)
        wait_in(b)

        @pl.when(i > 0)
        def _():
            wait_out(b - 1)

        @pl.when(jnp.logical_and(i < _BPW - 1, par == 0))
        def _():
            start_in(b + 1, 1)

        @pl.when(jnp.logical_and(i < _BPW - 1, par == 1))
        def _():
            start_in(b + 1, 0)

        lax.fori_loop(0, _L + 1, make_token_fn(par), 0)

        @pl.when(par == 0)
        def _():
            start_out(b, 0)

        @pl.when(par == 1)
        def _():
            start_out(b, 1)

        return carry

    lax.fori_loop(0, _BPW, one_batch, 0)
    wait_out(base + _BPW - 1)


def kernel(x_true, x_pred, masks, ind_of_ind):
    w = pl.pallas_call(
        _w_table_body,
        out_shape=jax.ShapeDtypeStruct((_D, _D), jnp.float32),
    )(masks, ind_of_ind.astype(jnp.int32).reshape(1, _D))
    mesh = plsc.VectorSubcoreMesh(core_axis_name="c", subcore_axis_name="s")
    f = functools.partial(
        pl.kernel,
        mesh=mesh,
        compiler_params=pltpu.CompilerParams(needs_layout_passes=False),
        out_type=jax.ShapeDtypeStruct((_B, _L, _D), jnp.float32),
        scratch_types=[
            pltpu.VMEM((2 * _L, _D), jnp.float32),
            pltpu.VMEM((2 * _L, _D), jnp.float32),
            pltpu.VMEM((_D, _D), jnp.float32),
            pltpu.SemaphoreType.DMA,
            pltpu.SemaphoreType.DMA,
        ],
    )(_sc_body)
    return f(x_true, x_pred, w)
